# baseline (speedup 1.0000x reference)
import functools

import jax
import jax.numpy as jnp
from jax import lax
from jax.experimental import pallas as pl
from jax.experimental.pallas import tpu as pltpu
from jax.experimental.pallas import tpu_sc as plsc

_L = 16


def _make_scatter_kernel(n_rows: int, n_idx: int):
    info = plsc.get_sparse_core_info()
    nc, ns = info.num_cores, info.num_subcores
    nw = nc * ns
    chunk = ((n_rows + nw - 1) // nw + _L - 1) // _L * _L
    last = n_rows - chunk * (nw - 1)
    assert 0 < last <= chunk and last % 8 == 0 and chunk % 8 == 0
    n_vecs = n_idx // _L
    assert n_idx % _L == 0

    mesh = plsc.VectorSubcoreMesh(core_axis_name="c", subcore_axis_name="s")

    @functools.partial(
        pl.kernel,
        mesh=mesh,
        out_type=jax.ShapeDtypeStruct((n_rows,), jnp.int32),
        scratch_types=[
            pltpu.VMEM((n_idx,), jnp.int32),
            pltpu.VMEM((chunk,), jnp.int32),
        ],
        compiler_params=pltpu.CompilerParams(needs_layout_passes=False),
    )
    def scatter_kernel(idx_hbm, out_hbm, idx_v, block_v):
        wid = lax.axis_index("s") * nc + lax.axis_index("c")
        lo = wid * chunk
        iota = lax.iota(jnp.int32, _L)
        zeros = jnp.zeros((_L,), jnp.int32)

        pltpu.sync_copy(idx_hbm, idx_v)

        def zero_body(i, _):
            block_v[pl.ds(i * _L, _L)] = zeros
            return 0

        lax.fori_loop(0, chunk // _L, zero_body, 0)

        def scan_body(j, _):
            v = idx_v[pl.ds(j * _L, _L)]
            rel = v - lo
            mask = (rel >= 0) & (rel < chunk)
            loc = jnp.where(mask, rel, 0)
            vals = j * _L + iota
            plsc.store_scatter(block_v, [loc], vals, mask=mask)

            g = plsc.load_gather(block_v, [loc], mask=mask)
            bad = mask & (g < vals)

            def fix_cond(b):
                return jnp.any(b)

            def fix_body(b):
                plsc.store_scatter(block_v, [loc], vals, mask=b)
                g2 = plsc.load_gather(block_v, [loc], mask=mask)
                return mask & (g2 < vals)

            lax.while_loop(fix_cond, fix_body, bad)
            return 0

        lax.fori_loop(0, n_vecs, scan_body, 0)

        @pl.when(wid < nw - 1)
        def _():
            pltpu.sync_copy(block_v, out_hbm.at[pl.ds(lo, chunk)])

        @pl.when(wid == nw - 1)
        def _():
            pltpu.sync_copy(
                block_v.at[pl.ds(0, last)], out_hbm.at[pl.ds(lo, last)]
            )

    return scatter_kernel


def kernel(X, row_missing_idx):
    n_rows = X.shape[0]
    n_idx = row_missing_idx.shape[0]
    sk = _make_scatter_kernel(n_rows, n_idx)
    return sk(row_missing_idx.astype(jnp.int32))


# speedup vs baseline: 1.4210x; 1.4210x over previous
import functools

import jax
import jax.numpy as jnp
from jax import lax
from jax.experimental import pallas as pl
from jax.experimental.pallas import tpu as pltpu
from jax.experimental.pallas import tpu_sc as plsc

_L = 16


def _make_scatter_kernel(n_rows: int, n_idx: int):
    info = plsc.get_sparse_core_info()
    nc, ns = info.num_cores, info.num_subcores
    nw = nc * ns
    chunk = ((n_rows + nw - 1) // nw + _L - 1) // _L * _L
    last = n_rows - chunk * (nw - 1)
    assert 0 < last <= chunk and last % 8 == 0 and chunk % 8 == 0
    n_vecs = n_idx // _L
    assert n_idx % _L == 0

    mesh = plsc.VectorSubcoreMesh(core_axis_name="c", subcore_axis_name="s")

    @functools.partial(
        pl.kernel,
        mesh=mesh,
        out_type=jax.ShapeDtypeStruct((n_rows,), jnp.int32),
        scratch_types=[
            pltpu.VMEM((n_idx,), jnp.int32),
            pltpu.VMEM((chunk,), jnp.int32),
            pltpu.SemaphoreType.DMA,
        ],
        compiler_params=pltpu.CompilerParams(needs_layout_passes=False),
    )
    def scatter_kernel(idx_hbm, out_hbm, idx_v, block_v, sem):
        wid = lax.axis_index("s") * nc + lax.axis_index("c")
        lo = wid * chunk
        iota = lax.iota(jnp.int32, _L)
        zeros = jnp.zeros((_L,), jnp.int32)

        in_dma = pltpu.async_copy(idx_hbm, idx_v, sem)

        def zero_body(i, _):
            block_v[pl.ds(i * _L, _L)] = zeros
            return 0

        lax.fori_loop(0, chunk // _L, zero_body, 0, unroll=8)

        in_dma.wait()

        def scan_body(j, vals):
            v = idx_v[pl.ds(j * _L, _L)]
            rel = v - lo
            mask = (rel >= 0) & (rel < chunk)
            loc = jnp.where(mask, rel, 0)
            plsc.store_scatter(block_v, [loc], vals, mask=mask)
            return vals + _L

        lax.fori_loop(0, n_vecs, scan_body, iota, unroll=4)

        def verify_pass(_):
            def body(j, carry):
                flag, vals = carry
                v = idx_v[pl.ds(j * _L, _L)]
                rel = v - lo
                mask = (rel >= 0) & (rel < chunk)
                loc = jnp.where(mask, rel, 0)
                g = plsc.load_gather(block_v, [loc], mask=mask)
                bad = mask & (g < vals)
                plsc.store_scatter(block_v, [loc], vals, mask=bad)
                return flag | bad, vals + _L

            flag, _ = lax.fori_loop(
                0, n_vecs, body, (jnp.zeros((_L,), jnp.bool_), iota), unroll=4
            )
            return jnp.any(flag)

        lax.while_loop(lambda b: b, verify_pass, verify_pass(0))

        @pl.when(wid < nw - 1)
        def _():
            pltpu.sync_copy(block_v, out_hbm.at[pl.ds(lo, chunk)])

        @pl.when(wid == nw - 1)
        def _():
            pltpu.sync_copy(
                block_v.at[pl.ds(0, last)], out_hbm.at[pl.ds(lo, last)]
            )

    return scatter_kernel


def kernel(X, row_missing_idx):
    n_rows = X.shape[0]
    n_idx = row_missing_idx.shape[0]
    sk = _make_scatter_kernel(n_rows, n_idx)
    return sk(row_missing_idx.astype(jnp.int32))
